# baseline (speedup 1.0000x reference)
import functools

import jax
import jax.numpy as jnp
import numpy as np
from jax import lax
from jax.experimental import pallas as pl
from jax.experimental.pallas import tpu as pltpu
from jax.experimental.pallas import tpu_sc as plsc

KERNEL = 4
RADIUS_SCALE = 1.5
PARTICLE_RADIUS = 0.025
FILTER_EXTENT = float(np.float32(RADIUS_SCALE * 6 * PARTICLE_RADIUS))
RADIUS = FILTER_EXTENT / 2.0
R2 = RADIUS * RADIUS
INV_R = 1.0 / RADIUS
DT = 1.0 / 50.0
K_NBRS = 32
NCELL = KERNEL ** 3


def _knn_body(exclude_self, nsrc, qpos_ref, srcT_ref, idx_ref, mask_ref):
    bq = qpos_ref.shape[0]
    qx = qpos_ref[:, 0:1]
    qy = qpos_ref[:, 1:2]
    qz = qpos_ref[:, 2:3]
    sx = srcT_ref[0:1, :]
    sy = srcT_ref[1:2, :]
    sz = srcT_ref[2:3, :]
    dx = qx - sx
    dy = qy - sy
    dz = qz - sz
    d2 = dx * dx + dy * dy + dz * dz
    col = lax.broadcasted_iota(jnp.int32, (1, nsrc), 1)
    bigi = jnp.int32(1 << 30)
    key = (lax.bitcast_convert_type(d2, jnp.int32) & jnp.int32(~0xFFF)) | col
    key = jnp.where(d2 <= R2, key, bigi)
    if exclude_self:
        row = pl.program_id(0) * bq + lax.broadcasted_iota(jnp.int32, (bq, 1), 0)
        key = jnp.where(col == row, bigi, key)
    for t in range(K_NBRS):
        m = jnp.min(key, axis=1, keepdims=True)
        idx_ref[:, t:t + 1] = m & jnp.int32(0xFFF)
        mask_ref[:, t:t + 1] = (m < bigi).astype(jnp.float32)
        key = jnp.where(key == m, bigi, key)


def _knn(qpos, srcT, nsrc, exclude_self):
    nq = qpos.shape[0]
    bq = 256
    body = functools.partial(_knn_body, exclude_self, nsrc)
    return pl.pallas_call(
        body,
        grid=(nq // bq,),
        in_specs=[
            pl.BlockSpec((bq, 3), lambda i: (i, 0)),
            pl.BlockSpec((8, nsrc), lambda i: (0, 0)),
        ],
        out_specs=[
            pl.BlockSpec((bq, K_NBRS), lambda i: (i, 0)),
            pl.BlockSpec((bq, K_NBRS), lambda i: (i, 0)),
        ],
        out_shape=[
            jax.ShapeDtypeStruct((nq, K_NBRS), jnp.int32),
            jax.ShapeDtypeStruct((nq, K_NBRS), jnp.float32),
        ],
    )(qpos, srcT)


def _sc_gather(table, idx):
    btot = idx.shape[0]
    d = table.shape[1]
    info = plsc.get_sparse_core_info()
    nw = info.num_cores * info.num_subcores
    b_per_w = btot // nw
    nbuf = 4
    chunk = min(128, b_per_w // nbuf) or b_per_w
    n_groups = b_per_w // (nbuf * chunk)
    mesh = plsc.VectorSubcoreMesh(core_axis_name="c", subcore_axis_name="s")

    @functools.partial(
        pl.kernel,
        mesh=mesh,
        out_type=jax.ShapeDtypeStruct((btot, d), jnp.float32),
        scratch_types=[
            pltpu.VMEM((b_per_w,), jnp.int32),
        ] + [pltpu.VMEM((chunk, d), jnp.float32) for _ in range(nbuf)]
          + [pltpu.SemaphoreType.DMA for _ in range(nbuf)],
    )
    def gather_k(table_hbm, idx_hbm, out_hbm, idx_v, *bufsem):
        rows = bufsem[:nbuf]
        sems = bufsem[nbuf:]
        wid = lax.axis_index("s") * info.num_cores + lax.axis_index("c")
        base = wid * b_per_w
        pltpu.sync_copy(idx_hbm.at[pl.ds(base, b_per_w)], idx_v)

        def step(g, carry):
            o0 = g * nbuf * chunk
            cps = []
            for b in range(nbuf):
                ob = o0 + b * chunk
                cps.append(pltpu.async_copy(
                    table_hbm.at[idx_v.at[pl.ds(ob, chunk)]], rows[b], sems[b]))
            for b in range(nbuf):
                ob = o0 + b * chunk
                cps[b].wait()
                pltpu.sync_copy(rows[b], out_hbm.at[pl.ds(base + ob, chunk)])
            return carry

        lax.fori_loop(0, n_groups, step, 0)

    return gather_k(table, idx)


def _accbuild_body(qpos_ref, nbrx_ref, nbry_ref, nbrz_ref, maskf_ref, fg_ref,
                   acc_ref):
    bq = qpos_ref.shape[0]
    ox = (nbrx_ref[...] - qpos_ref[:, 0:1]) * INV_R
    oy = (nbry_ref[...] - qpos_ref[:, 1:2]) * INV_R
    oz = (nbrz_ref[...] - qpos_ref[:, 2:3]) * INV_R
    r2 = ox * ox + oy * oy + oz * oz
    win = jnp.clip((1.0 - r2) ** 3, 0.0, 1.0) * maskf_ref[...]
    norm2 = jnp.sqrt(jnp.maximum(r2, 1e-24))
    norminf = jnp.maximum(
        jnp.maximum(jnp.abs(ox), jnp.maximum(jnp.abs(oy), jnp.abs(oz))),
        1e-12)
    scale = norm2 / norminf

    def uax(o):
        u = jnp.clip((o * scale * 0.5 + 0.5) * (KERNEL - 1), 0.0,
                     float(KERNEL - 1))
        u0 = jnp.clip(jnp.floor(u), 0.0, float(KERNEL - 2))
        return u0.astype(jnp.int32), u - u0

    u0x, fx = uax(ox)
    u0y, fy = uax(oy)
    u0z, fz = uax(oz)
    base = (u0x * KERNEL + u0y) * KERNEL + u0z
    cell_iota = lax.broadcasted_iota(jnp.int32, (1, 1, NCELL), 2)
    e = jnp.zeros((bq, K_NBRS, NCELL), jnp.float32)
    for dxi in (0, 1):
        wx = fx if dxi else (1.0 - fx)
        for dyi in (0, 1):
            wy = fy if dyi else (1.0 - fy)
            for dzi in (0, 1):
                wz = fz if dzi else (1.0 - fz)
                c = base + jnp.int32(dxi * KERNEL * KERNEL + dyi * KERNEL + dzi)
                e = e + jnp.where(cell_iota == c[:, :, None],
                                  (wx * wy * wz * win)[:, :, None], 0.0)
    acc_ref[...] = lax.dot_general(
        e, fg_ref[...],
        dimension_numbers=(((1,), (1,)), ((0,), (0,))),
        preferred_element_type=jnp.float32)


def _accbuild(qpos, nbrx, nbry, nbrz, maskf, fg):
    n = qpos.shape[0]
    cinp = fg.shape[2]
    bq = 128
    return pl.pallas_call(
        _accbuild_body,
        grid=(n // bq,),
        in_specs=[
            pl.BlockSpec((bq, 3), lambda i: (i, 0)),
            pl.BlockSpec((bq, K_NBRS), lambda i: (i, 0)),
            pl.BlockSpec((bq, K_NBRS), lambda i: (i, 0)),
            pl.BlockSpec((bq, K_NBRS), lambda i: (i, 0)),
            pl.BlockSpec((bq, K_NBRS), lambda i: (i, 0)),
            pl.BlockSpec((bq, K_NBRS, cinp), lambda i: (i, 0, 0)),
        ],
        out_specs=pl.BlockSpec((bq, NCELL, cinp), lambda i: (i, 0, 0)),
        out_shape=jax.ShapeDtypeStruct((n, NCELL, cinp), jnp.float32),
    )(qpos, nbrx, nbry, nbrz, maskf, fg)


def _conv0_body(accO_ref, accF_ref, f0_ref, WcO_ref, WcF_ref, Wd0_ref, b_ref,
                x1_ref, y1_ref):
    oO = jnp.dot(accO_ref[...], WcO_ref[...],
                 preferred_element_type=jnp.float32) + b_ref[0:1, 0:32]
    oF = jnp.dot(accF_ref[...], WcF_ref[...],
                 preferred_element_type=jnp.float32) + b_ref[0:1, 32:64]
    oD = jnp.dot(f0_ref[...], Wd0_ref[...],
                 preferred_element_type=jnp.float32) + b_ref[0:1, 64:96]
    x1 = jnp.concatenate([oO, oF, oD], axis=1)
    x1_ref[...] = x1
    y1_ref[...] = jnp.maximum(x1, 0.0)


def _conv0(accO2, accF2, f0, WcO, WcF, Wd0, b96):
    n = f0.shape[0]
    bq = 256
    return pl.pallas_call(
        _conv0_body,
        grid=(n // bq,),
        in_specs=[
            pl.BlockSpec((bq, accO2.shape[1]), lambda i: (i, 0)),
            pl.BlockSpec((bq, accF2.shape[1]), lambda i: (i, 0)),
            pl.BlockSpec((bq, 4), lambda i: (i, 0)),
            pl.BlockSpec(WcO.shape, lambda i: (0, 0)),
            pl.BlockSpec(WcF.shape, lambda i: (0, 0)),
            pl.BlockSpec(Wd0.shape, lambda i: (0, 0)),
            pl.BlockSpec((1, 96), lambda i: (0, 0)),
        ],
        out_specs=[
            pl.BlockSpec((bq, 96), lambda i: (i, 0)),
            pl.BlockSpec((bq, 96), lambda i: (i, 0)),
        ],
        out_shape=[
            jax.ShapeDtypeStruct((n, 96), jnp.float32),
            jax.ShapeDtypeStruct((n, 96), jnp.float32),
        ],
    )(accO2, accF2, f0, WcO, WcF, Wd0, b96)


def _mid_body(residual, acc_ref, y_ref, x_ref, Wc_ref, Wd_ref, b_ref,
              xn_ref, yn_ref):
    o = jnp.dot(acc_ref[...], Wc_ref[...], preferred_element_type=jnp.float32)
    o = o + jnp.dot(y_ref[...], Wd_ref[...],
                    preferred_element_type=jnp.float32) + b_ref[...]
    if residual:
        o = o + x_ref[...]
    xn_ref[...] = o
    yn_ref[...] = jnp.maximum(o, 0.0)


def _mid(acc2, y, x, Wc, Wd, b, residual):
    n = y.shape[0]
    bq = 256
    cout = Wd.shape[1]
    body = functools.partial(_mid_body, residual)
    return pl.pallas_call(
        body,
        grid=(n // bq,),
        in_specs=[
            pl.BlockSpec((bq, acc2.shape[1]), lambda i: (i, 0)),
            pl.BlockSpec((bq, y.shape[1]), lambda i: (i, 0)),
            pl.BlockSpec((bq, x.shape[1]), lambda i: (i, 0)),
            pl.BlockSpec(Wc.shape, lambda i: (0, 0)),
            pl.BlockSpec(Wd.shape, lambda i: (0, 0)),
            pl.BlockSpec((1, cout), lambda i: (0, 0)),
        ],
        out_specs=[
            pl.BlockSpec((bq, cout), lambda i: (i, 0)),
            pl.BlockSpec((bq, cout), lambda i: (i, 0)),
        ],
        out_shape=[
            jax.ShapeDtypeStruct((n, cout), jnp.float32),
            jax.ShapeDtypeStruct((n, cout), jnp.float32),
        ],
    )(acc2, y, x, Wc, Wd, b)


def _final_body(acc_ref, y_ref, pos2_ref, pos_ref, Wc_ref, Wd_ref, b_ref,
                pn_ref, vn_ref):
    o = jnp.dot(acc_ref[...], Wc_ref[...], preferred_element_type=jnp.float32)
    o = o + jnp.dot(y_ref[...], Wd_ref[...],
                    preferred_element_type=jnp.float32) + b_ref[...]
    pn = pos2_ref[...] + o * (1.0 / 128.0)
    pn_ref[...] = pn
    vn_ref[...] = (pn - pos_ref[...]) * (1.0 / DT)


def _final(acc2, y, pos2, pos, Wc, Wd, b):
    n = y.shape[0]
    bq = 256
    return pl.pallas_call(
        _final_body,
        grid=(n // bq,),
        in_specs=[
            pl.BlockSpec((bq, acc2.shape[1]), lambda i: (i, 0)),
            pl.BlockSpec((bq, y.shape[1]), lambda i: (i, 0)),
            pl.BlockSpec((bq, 3), lambda i: (i, 0)),
            pl.BlockSpec((bq, 3), lambda i: (i, 0)),
            pl.BlockSpec(Wc.shape, lambda i: (0, 0)),
            pl.BlockSpec(Wd.shape, lambda i: (0, 0)),
            pl.BlockSpec((1, 3), lambda i: (0, 0)),
        ],
        out_specs=[
            pl.BlockSpec((bq, 3), lambda i: (i, 0)),
            pl.BlockSpec((bq, 3), lambda i: (i, 0)),
        ],
        out_shape=[
            jax.ShapeDtypeStruct((n, 3), jnp.float32),
            jax.ShapeDtypeStruct((n, 3), jnp.float32),
        ],
    )(acc2, y, pos2, pos, Wc, Wd, b)


def kernel(pos, vel, box, box_feats,
           Wc0f, bc0f, Wc0o, bc0o, Wd0, bd0,
           Wd1, bd1, Wc1, bc1, Wd2, bd2, Wc2, bc2, Wd3, bd3, Wc3, bc3):
    n = pos.shape[0]
    m = box.shape[0]
    gravity = jnp.array([0.0, -9.81, 0.0], dtype=jnp.float32)
    vel2 = vel + DT * gravity
    pos2 = pos + DT * (vel2 + vel) / 2.0
    feats0 = jnp.concatenate([jnp.ones((n, 1), jnp.float32), vel2], axis=1)

    fluid_t = jnp.pad(jnp.concatenate([pos2, feats0], axis=1),
                      ((0, 0), (0, 121)))
    box_t = jnp.pad(jnp.concatenate([box, box_feats], axis=1),
                    ((0, 0), (0, 122)))
    pos2T = jnp.pad(pos2.T, ((0, 5), (0, 0)))
    boxT = jnp.pad(box.T, ((0, 5), (0, 0)))

    idxF, maskF = _knn(pos2, pos2T, n, True)
    idxO, maskO = _knn(pos2, boxT, m, False)
    idxF_flat = idxF.reshape(-1)
    idxO_flat = idxO.reshape(-1)
    gF = _sc_gather(fluid_t, idxF_flat)
    gO = _sc_gather(box_t, idxO_flat)
    nbrxF = gF[:, 0].reshape(n, K_NBRS)
    nbryF = gF[:, 1].reshape(n, K_NBRS)
    nbrzF = gF[:, 2].reshape(n, K_NBRS)
    nbrxO = gO[:, 0].reshape(n, K_NBRS)
    nbryO = gO[:, 1].reshape(n, K_NBRS)
    nbrzO = gO[:, 2].reshape(n, K_NBRS)

    fgF = gF[:, 3:7].reshape(n, K_NBRS, 4)
    fgO = gO[:, 3:6].reshape(n, K_NBRS, 3)
    accF = _accbuild(pos2, nbrxF, nbryF, nbrzF, maskF, fgF).reshape(n, NCELL * 4)
    accO = _accbuild(pos2, nbrxO, nbryO, nbrzO, maskO, fgO).reshape(n, NCELL * 3)
    WcF_flat = Wc0f.reshape(NCELL * 4, 32)
    WcO_flat = Wc0o.reshape(NCELL * 3, 32)
    b96 = jnp.concatenate([bc0o, bc0f, bd0]).reshape(1, 96)
    x1, y1 = _conv0(accO, accF, feats0, WcO_flat, WcF_flat, Wd0, b96)

    y1_t = jnp.pad(y1, ((0, 0), (0, 32)))
    fg1 = _sc_gather(y1_t, idxF_flat)[:, :96].reshape(n, K_NBRS, 96)
    acc1 = _accbuild(pos2, nbrxF, nbryF, nbrzF, maskF, fg1).reshape(n, NCELL * 96)
    x2, y2 = _mid(acc1, y1, y1, Wc1.reshape(NCELL * 96, 64), Wd1,
                  (bc1 + bd1).reshape(1, 64), residual=False)

    y2_t = jnp.pad(y2, ((0, 0), (0, 64)))
    fg2 = _sc_gather(y2_t, idxF_flat)[:, :64].reshape(n, K_NBRS, 64)
    acc2 = _accbuild(pos2, nbrxF, nbryF, nbrzF, maskF, fg2).reshape(n, NCELL * 64)
    x3, y3 = _mid(acc2, y2, x2, Wc2.reshape(NCELL * 64, 64), Wd2,
                  (bc2 + bd2).reshape(1, 64), residual=True)

    y3_t = jnp.pad(y3, ((0, 0), (0, 64)))
    fg3 = _sc_gather(y3_t, idxF_flat)[:, :64].reshape(n, K_NBRS, 64)
    acc3 = _accbuild(pos2, nbrxF, nbryF, nbrzF, maskF, fg3).reshape(n, NCELL * 64)
    pos_new, vel_new = _final(acc3, y3, pos2, pos, Wc3.reshape(NCELL * 64, 3),
                              Wd3, (bc3 + bd3).reshape(1, 3))
    return pos_new, vel_new


# speedup vs baseline: 1.9633x; 1.9633x over previous
import functools

import jax
import jax.numpy as jnp
import numpy as np
from jax import lax
from jax.experimental import pallas as pl
from jax.experimental.pallas import tpu as pltpu
from jax.experimental.pallas import tpu_sc as plsc

KERNEL = 4
RADIUS_SCALE = 1.5
PARTICLE_RADIUS = 0.025
FILTER_EXTENT = float(np.float32(RADIUS_SCALE * 6 * PARTICLE_RADIUS))
RADIUS = FILTER_EXTENT / 2.0
R2 = RADIUS * RADIUS
INV_R = 1.0 / RADIUS
DT = 1.0 / 50.0
K_NBRS = 32
NCELL = KERNEL ** 3


def _knn_body(exclude_self, nsrc, qpos_ref, srcT_ref, idx_ref, mask_ref):
    bq = qpos_ref.shape[0]
    qx = qpos_ref[:, 0:1]
    qy = qpos_ref[:, 1:2]
    qz = qpos_ref[:, 2:3]
    sx = srcT_ref[0:1, :]
    sy = srcT_ref[1:2, :]
    sz = srcT_ref[2:3, :]
    dx = qx - sx
    dy = qy - sy
    dz = qz - sz
    d2 = dx * dx + dy * dy + dz * dz
    col = lax.broadcasted_iota(jnp.int32, (1, nsrc), 1)
    bigi = jnp.int32(1 << 30)
    key = (lax.bitcast_convert_type(d2, jnp.int32) & jnp.int32(~0xFFF)) | col
    key = jnp.where(d2 <= R2, key, bigi)
    if exclude_self:
        row = pl.program_id(0) * bq + lax.broadcasted_iota(jnp.int32, (bq, 1), 0)
        key = jnp.where(col == row, bigi, key)
    for t in range(K_NBRS):
        m = jnp.min(key, axis=1, keepdims=True)
        idx_ref[:, t:t + 1] = m & jnp.int32(0xFFF)
        mask_ref[:, t:t + 1] = (m < bigi).astype(jnp.float32)
        key = jnp.where(key == m, bigi, key)


def _knn(qpos, srcT, nsrc, exclude_self):
    nq = qpos.shape[0]
    bq = 256
    body = functools.partial(_knn_body, exclude_self, nsrc)
    return pl.pallas_call(
        body,
        grid=(nq // bq,),
        in_specs=[
            pl.BlockSpec((bq, 3), lambda i: (i, 0)),
            pl.BlockSpec((8, nsrc), lambda i: (0, 0)),
        ],
        out_specs=[
            pl.BlockSpec((bq, K_NBRS), lambda i: (i, 0)),
            pl.BlockSpec((bq, K_NBRS), lambda i: (i, 0)),
        ],
        out_shape=[
            jax.ShapeDtypeStruct((nq, K_NBRS), jnp.int32),
            jax.ShapeDtypeStruct((nq, K_NBRS), jnp.float32),
        ],
    )(qpos, srcT)


def _sc_gather(table, idx):
    btot = idx.shape[0]
    d = table.shape[1]
    dt = table.dtype
    info = plsc.get_sparse_core_info()
    nw = info.num_cores * info.num_subcores
    b_per_w = btot // nw
    nbuf = 4
    chunk = min(128, b_per_w // nbuf) or b_per_w
    n_groups = b_per_w // (nbuf * chunk)
    mesh = plsc.VectorSubcoreMesh(core_axis_name="c", subcore_axis_name="s")

    @functools.partial(
        pl.kernel,
        mesh=mesh,
        compiler_params=pltpu.CompilerParams(use_tc_tiling_on_sc=False),
        out_type=jax.ShapeDtypeStruct((btot, d), dt),
        scratch_types=[
            pltpu.VMEM((b_per_w,), jnp.int32),
        ] + [pltpu.VMEM((chunk, d), dt) for _ in range(nbuf)]
          + [pltpu.SemaphoreType.DMA for _ in range(nbuf)],
    )
    def gather_k(table_hbm, idx_hbm, out_hbm, idx_v, *bufsem):
        rows = bufsem[:nbuf]
        sems = bufsem[nbuf:]
        wid = lax.axis_index("s") * info.num_cores + lax.axis_index("c")
        base = wid * b_per_w
        pltpu.sync_copy(idx_hbm.at[pl.ds(base, b_per_w)], idx_v)

        def step(g, carry):
            o0 = g * nbuf * chunk
            cps = []
            for b in range(nbuf):
                ob = o0 + b * chunk
                cps.append(pltpu.async_copy(
                    table_hbm.at[idx_v.at[pl.ds(ob, chunk)]], rows[b], sems[b]))
            for b in range(nbuf):
                ob = o0 + b * chunk
                cps[b].wait()
                pltpu.sync_copy(rows[b], out_hbm.at[pl.ds(base + ob, chunk)])
            return carry

        lax.fori_loop(0, n_groups, step, 0)

    return gather_k(table, idx)


def _accbuild_body(qpos_ref, nbrx_ref, nbry_ref, nbrz_ref, maskf_ref, fg_ref,
                   acc_ref):
    bq = qpos_ref.shape[0]
    ox = (nbrx_ref[...] - qpos_ref[:, 0:1]) * INV_R
    oy = (nbry_ref[...] - qpos_ref[:, 1:2]) * INV_R
    oz = (nbrz_ref[...] - qpos_ref[:, 2:3]) * INV_R
    r2 = ox * ox + oy * oy + oz * oz
    win = jnp.clip((1.0 - r2) ** 3, 0.0, 1.0) * maskf_ref[...]
    norm2 = jnp.sqrt(jnp.maximum(r2, 1e-24))
    norminf = jnp.maximum(
        jnp.maximum(jnp.abs(ox), jnp.maximum(jnp.abs(oy), jnp.abs(oz))),
        1e-12)
    scale = norm2 / norminf

    def uax(o):
        u = jnp.clip((o * scale * 0.5 + 0.5) * (KERNEL - 1), 0.0,
                     float(KERNEL - 1))
        u0 = jnp.clip(jnp.floor(u), 0.0, float(KERNEL - 2))
        return u0.astype(jnp.int32), u - u0

    u0x, fx = uax(ox)
    u0y, fy = uax(oy)
    u0z, fz = uax(oz)
    base = (u0x * KERNEL + u0y) * KERNEL + u0z
    cell_iota = lax.broadcasted_iota(jnp.int32, (1, 1, NCELL), 2)
    e = jnp.zeros((bq, K_NBRS, NCELL), jnp.float32)
    for dxi in (0, 1):
        wx = fx if dxi else (1.0 - fx)
        for dyi in (0, 1):
            wy = fy if dyi else (1.0 - fy)
            for dzi in (0, 1):
                wz = fz if dzi else (1.0 - fz)
                c = base + jnp.int32(dxi * KERNEL * KERNEL + dyi * KERNEL + dzi)
                e = e + jnp.where(cell_iota == c[:, :, None],
                                  (wx * wy * wz * win)[:, :, None], 0.0)
    acc_ref[...] = lax.dot_general(
        e, fg_ref[...].astype(jnp.float32),
        dimension_numbers=(((1,), (1,)), ((0,), (0,))),
        preferred_element_type=jnp.float32)


def _accbuild(qpos, nbrx, nbry, nbrz, maskf, fg):
    n = qpos.shape[0]
    cinp = fg.shape[2]
    bq = 128
    return pl.pallas_call(
        _accbuild_body,
        grid=(n // bq,),
        in_specs=[
            pl.BlockSpec((bq, 3), lambda i: (i, 0)),
            pl.BlockSpec((bq, K_NBRS), lambda i: (i, 0)),
            pl.BlockSpec((bq, K_NBRS), lambda i: (i, 0)),
            pl.BlockSpec((bq, K_NBRS), lambda i: (i, 0)),
            pl.BlockSpec((bq, K_NBRS), lambda i: (i, 0)),
            pl.BlockSpec((bq, K_NBRS, cinp), lambda i: (i, 0, 0)),
        ],
        out_specs=pl.BlockSpec((bq, NCELL, cinp), lambda i: (i, 0, 0)),
        out_shape=jax.ShapeDtypeStruct((n, NCELL, cinp), jnp.float32),
    )(qpos, nbrx, nbry, nbrz, maskf, fg)


def _conv0_body(accO_ref, accF_ref, f0_ref, WcO_ref, WcF_ref, Wd0_ref, b_ref,
                x1_ref, y1_ref):
    oO = jnp.dot(accO_ref[...], WcO_ref[...],
                 preferred_element_type=jnp.float32) + b_ref[0:1, 0:32]
    oF = jnp.dot(accF_ref[...], WcF_ref[...],
                 preferred_element_type=jnp.float32) + b_ref[0:1, 32:64]
    oD = jnp.dot(f0_ref[...], Wd0_ref[...],
                 preferred_element_type=jnp.float32) + b_ref[0:1, 64:96]
    x1 = jnp.concatenate([oO, oF, oD], axis=1)
    x1_ref[...] = x1
    y1_ref[...] = jnp.maximum(x1, 0.0)


def _conv0(accO2, accF2, f0, WcO, WcF, Wd0, b96):
    n = f0.shape[0]
    bq = 256
    return pl.pallas_call(
        _conv0_body,
        grid=(n // bq,),
        in_specs=[
            pl.BlockSpec((bq, accO2.shape[1]), lambda i: (i, 0)),
            pl.BlockSpec((bq, accF2.shape[1]), lambda i: (i, 0)),
            pl.BlockSpec((bq, 4), lambda i: (i, 0)),
            pl.BlockSpec(WcO.shape, lambda i: (0, 0)),
            pl.BlockSpec(WcF.shape, lambda i: (0, 0)),
            pl.BlockSpec(Wd0.shape, lambda i: (0, 0)),
            pl.BlockSpec((1, 96), lambda i: (0, 0)),
        ],
        out_specs=[
            pl.BlockSpec((bq, 96), lambda i: (i, 0)),
            pl.BlockSpec((bq, 96), lambda i: (i, 0)),
        ],
        out_shape=[
            jax.ShapeDtypeStruct((n, 96), jnp.float32),
            jax.ShapeDtypeStruct((n, 96), jnp.float32),
        ],
    )(accO2, accF2, f0, WcO, WcF, Wd0, b96)


def _mid_body(residual, acc_ref, y_ref, x_ref, Wc_ref, Wd_ref, b_ref,
              xn_ref, yn_ref):
    o = jnp.dot(acc_ref[...], Wc_ref[...], preferred_element_type=jnp.float32)
    o = o + jnp.dot(y_ref[...], Wd_ref[...],
                    preferred_element_type=jnp.float32) + b_ref[...]
    if residual:
        o = o + x_ref[...]
    xn_ref[...] = o
    yn_ref[...] = jnp.maximum(o, 0.0)


def _mid(acc2, y, x, Wc, Wd, b, residual):
    n = y.shape[0]
    bq = 256
    cout = Wd.shape[1]
    body = functools.partial(_mid_body, residual)
    return pl.pallas_call(
        body,
        grid=(n // bq,),
        in_specs=[
            pl.BlockSpec((bq, acc2.shape[1]), lambda i: (i, 0)),
            pl.BlockSpec((bq, y.shape[1]), lambda i: (i, 0)),
            pl.BlockSpec((bq, x.shape[1]), lambda i: (i, 0)),
            pl.BlockSpec(Wc.shape, lambda i: (0, 0)),
            pl.BlockSpec(Wd.shape, lambda i: (0, 0)),
            pl.BlockSpec((1, cout), lambda i: (0, 0)),
        ],
        out_specs=[
            pl.BlockSpec((bq, cout), lambda i: (i, 0)),
            pl.BlockSpec((bq, cout), lambda i: (i, 0)),
        ],
        out_shape=[
            jax.ShapeDtypeStruct((n, cout), jnp.float32),
            jax.ShapeDtypeStruct((n, cout), jnp.float32),
        ],
    )(acc2, y, x, Wc, Wd, b)


def _final_body(acc_ref, y_ref, pos2_ref, pos_ref, Wc_ref, Wd_ref, b_ref,
                pn_ref, vn_ref):
    o = jnp.dot(acc_ref[...], Wc_ref[...], preferred_element_type=jnp.float32)
    o = o + jnp.dot(y_ref[...], Wd_ref[...],
                    preferred_element_type=jnp.float32) + b_ref[...]
    pn = pos2_ref[...] + o * (1.0 / 128.0)
    pn_ref[...] = pn
    vn_ref[...] = (pn - pos_ref[...]) * (1.0 / DT)


def _final(acc2, y, pos2, pos, Wc, Wd, b):
    n = y.shape[0]
    bq = 256
    return pl.pallas_call(
        _final_body,
        grid=(n // bq,),
        in_specs=[
            pl.BlockSpec((bq, acc2.shape[1]), lambda i: (i, 0)),
            pl.BlockSpec((bq, y.shape[1]), lambda i: (i, 0)),
            pl.BlockSpec((bq, 3), lambda i: (i, 0)),
            pl.BlockSpec((bq, 3), lambda i: (i, 0)),
            pl.BlockSpec(Wc.shape, lambda i: (0, 0)),
            pl.BlockSpec(Wd.shape, lambda i: (0, 0)),
            pl.BlockSpec((1, 3), lambda i: (0, 0)),
        ],
        out_specs=[
            pl.BlockSpec((bq, 3), lambda i: (i, 0)),
            pl.BlockSpec((bq, 3), lambda i: (i, 0)),
        ],
        out_shape=[
            jax.ShapeDtypeStruct((n, 3), jnp.float32),
            jax.ShapeDtypeStruct((n, 3), jnp.float32),
        ],
    )(acc2, y, pos2, pos, Wc, Wd, b)


def kernel(pos, vel, box, box_feats,
           Wc0f, bc0f, Wc0o, bc0o, Wd0, bd0,
           Wd1, bd1, Wc1, bc1, Wd2, bd2, Wc2, bc2, Wd3, bd3, Wc3, bc3):
    n = pos.shape[0]
    m = box.shape[0]
    gravity = jnp.array([0.0, -9.81, 0.0], dtype=jnp.float32)
    vel2 = vel + DT * gravity
    pos2 = pos + DT * (vel2 + vel) / 2.0
    feats0 = jnp.concatenate([jnp.ones((n, 1), jnp.float32), vel2], axis=1)

    fluid_t = jnp.pad(jnp.concatenate([pos2, feats0], axis=1),
                      ((0, 0), (0, 9)))
    box_t = jnp.pad(jnp.concatenate([box, box_feats], axis=1),
                    ((0, 0), (0, 10)))
    pos2T = jnp.pad(pos2.T, ((0, 5), (0, 0)))
    boxT = jnp.pad(box.T, ((0, 5), (0, 0)))

    idxF, maskF = _knn(pos2, pos2T, n, True)
    idxO, maskO = _knn(pos2, boxT, m, False)
    idxF_flat = idxF.reshape(-1)
    idxO_flat = idxO.reshape(-1)
    gF = _sc_gather(fluid_t, idxF_flat)
    gO = _sc_gather(box_t, idxO_flat)
    nbrxF = gF[:, 0].reshape(n, K_NBRS)
    nbryF = gF[:, 1].reshape(n, K_NBRS)
    nbrzF = gF[:, 2].reshape(n, K_NBRS)
    nbrxO = gO[:, 0].reshape(n, K_NBRS)
    nbryO = gO[:, 1].reshape(n, K_NBRS)
    nbrzO = gO[:, 2].reshape(n, K_NBRS)

    fgF = gF[:, 3:7].reshape(n, K_NBRS, 4)
    fgO = gO[:, 3:6].reshape(n, K_NBRS, 3)
    accF = _accbuild(pos2, nbrxF, nbryF, nbrzF, maskF, fgF).reshape(n, NCELL * 4)
    accO = _accbuild(pos2, nbrxO, nbryO, nbrzO, maskO, fgO).reshape(n, NCELL * 3)
    WcF_flat = Wc0f.reshape(NCELL * 4, 32)
    WcO_flat = Wc0o.reshape(NCELL * 3, 32)
    b96 = jnp.concatenate([bc0o, bc0f, bd0]).reshape(1, 96)
    x1, y1 = _conv0(accO, accF, feats0, WcO_flat, WcF_flat, Wd0, b96)

    fg1 = _sc_gather(y1, idxF_flat).reshape(n, K_NBRS, 96)
    acc1 = _accbuild(pos2, nbrxF, nbryF, nbrzF, maskF, fg1).reshape(n, NCELL * 96)
    x2, y2 = _mid(acc1, y1, y1, Wc1.reshape(NCELL * 96, 64), Wd1,
                  (bc1 + bd1).reshape(1, 64), residual=False)

    fg2 = _sc_gather(y2, idxF_flat).reshape(n, K_NBRS, 64)
    acc2 = _accbuild(pos2, nbrxF, nbryF, nbrzF, maskF, fg2).reshape(n, NCELL * 64)
    x3, y3 = _mid(acc2, y2, x2, Wc2.reshape(NCELL * 64, 64), Wd2,
                  (bc2 + bd2).reshape(1, 64), residual=True)

    fg3 = _sc_gather(y3, idxF_flat).reshape(n, K_NBRS, 64)
    acc3 = _accbuild(pos2, nbrxF, nbryF, nbrzF, maskF, fg3).reshape(n, NCELL * 64)
    pos_new, vel_new = _final(acc3, y3, pos2, pos, Wc3.reshape(NCELL * 64, 3),
                              Wd3, (bc3 + bd3).reshape(1, 3))
    return pos_new, vel_new


# speedup vs baseline: 1.9646x; 1.0006x over previous
import functools

import jax
import jax.numpy as jnp
import numpy as np
from jax import lax
from jax.experimental import pallas as pl
from jax.experimental.pallas import tpu as pltpu
from jax.experimental.pallas import tpu_sc as plsc

KERNEL = 4
RADIUS_SCALE = 1.5
PARTICLE_RADIUS = 0.025
FILTER_EXTENT = float(np.float32(RADIUS_SCALE * 6 * PARTICLE_RADIUS))
RADIUS = FILTER_EXTENT / 2.0
R2 = RADIUS * RADIUS
INV_R = 1.0 / RADIUS
DT = 1.0 / 50.0
K_NBRS = 32
NCELL = KERNEL ** 3


def _knn_body(exclude_self, nsrc, qpos_ref, srcT_ref, idx_ref, mask_ref):
    bq = qpos_ref.shape[0]
    qx = qpos_ref[:, 0:1]
    qy = qpos_ref[:, 1:2]
    qz = qpos_ref[:, 2:3]
    sx = srcT_ref[0:1, :]
    sy = srcT_ref[1:2, :]
    sz = srcT_ref[2:3, :]
    dx = qx - sx
    dy = qy - sy
    dz = qz - sz
    d2 = dx * dx + dy * dy + dz * dz
    col = lax.broadcasted_iota(jnp.int32, (1, nsrc), 1)
    bigi = jnp.int32(1 << 30)
    key = (lax.bitcast_convert_type(d2, jnp.int32) & jnp.int32(~0xFFF)) | col
    key = jnp.where(d2 <= R2, key, bigi)
    if exclude_self:
        row = pl.program_id(0) * bq + lax.broadcasted_iota(jnp.int32, (bq, 1), 0)
        key = jnp.where(col == row, bigi, key)
    for t in range(K_NBRS):
        m = jnp.min(key, axis=1, keepdims=True)
        idx_ref[:, t:t + 1] = m & jnp.int32(0xFFF)
        mask_ref[:, t:t + 1] = (m < bigi).astype(jnp.float32)
        key = jnp.where(key == m, bigi, key)


def _knn(qpos, srcT, nsrc, exclude_self):
    nq = qpos.shape[0]
    bq = 256
    body = functools.partial(_knn_body, exclude_self, nsrc)
    return pl.pallas_call(
        body,
        grid=(nq // bq,),
        in_specs=[
            pl.BlockSpec((bq, 3), lambda i: (i, 0)),
            pl.BlockSpec((8, nsrc), lambda i: (0, 0)),
        ],
        out_specs=[
            pl.BlockSpec((bq, K_NBRS), lambda i: (i, 0)),
            pl.BlockSpec((bq, K_NBRS), lambda i: (i, 0)),
        ],
        out_shape=[
            jax.ShapeDtypeStruct((nq, K_NBRS), jnp.int32),
            jax.ShapeDtypeStruct((nq, K_NBRS), jnp.float32),
        ],
    )(qpos, srcT)


def _sc_gather(table, idx):
    btot = idx.shape[0]
    d = table.shape[1]
    dt = table.dtype
    info = plsc.get_sparse_core_info()
    nw = info.num_cores * info.num_subcores
    b_per_w = btot // nw
    nbuf = 4
    byte_cap = 300 * 1024 // (nbuf * d * 4)
    chunk = min(512, b_per_w // nbuf, max(128, byte_cap // 128 * 128))
    n_groups = b_per_w // (nbuf * chunk)
    mesh = plsc.VectorSubcoreMesh(core_axis_name="c", subcore_axis_name="s")

    @functools.partial(
        pl.kernel,
        mesh=mesh,
        compiler_params=pltpu.CompilerParams(use_tc_tiling_on_sc=False),
        out_type=jax.ShapeDtypeStruct((btot, d), dt),
        scratch_types=[
            pltpu.VMEM((b_per_w,), jnp.int32),
        ] + [pltpu.VMEM((chunk, d), dt) for _ in range(nbuf)]
          + [pltpu.SemaphoreType.DMA for _ in range(nbuf)],
    )
    def gather_k(table_hbm, idx_hbm, out_hbm, idx_v, *bufsem):
        rows = bufsem[:nbuf]
        sems = bufsem[nbuf:]
        wid = lax.axis_index("s") * info.num_cores + lax.axis_index("c")
        base = wid * b_per_w
        pltpu.sync_copy(idx_hbm.at[pl.ds(base, b_per_w)], idx_v)

        def step(g, carry):
            o0 = g * nbuf * chunk
            cps = []
            for b in range(nbuf):
                ob = o0 + b * chunk
                cps.append(pltpu.async_copy(
                    table_hbm.at[idx_v.at[pl.ds(ob, chunk)]], rows[b], sems[b]))
            for b in range(nbuf):
                ob = o0 + b * chunk
                cps[b].wait()
                pltpu.sync_copy(rows[b], out_hbm.at[pl.ds(base + ob, chunk)])
            return carry

        lax.fori_loop(0, n_groups, step, 0)

    return gather_k(table, idx)


def _accbuild_body(qpos_ref, nbrx_ref, nbry_ref, nbrz_ref, maskf_ref, fg_ref,
                   acc_ref):
    bq = qpos_ref.shape[0]
    ox = (nbrx_ref[...] - qpos_ref[:, 0:1]) * INV_R
    oy = (nbry_ref[...] - qpos_ref[:, 1:2]) * INV_R
    oz = (nbrz_ref[...] - qpos_ref[:, 2:3]) * INV_R
    r2 = ox * ox + oy * oy + oz * oz
    win = jnp.clip((1.0 - r2) ** 3, 0.0, 1.0) * maskf_ref[...]
    norm2 = jnp.sqrt(jnp.maximum(r2, 1e-24))
    norminf = jnp.maximum(
        jnp.maximum(jnp.abs(ox), jnp.maximum(jnp.abs(oy), jnp.abs(oz))),
        1e-12)
    scale = norm2 / norminf

    def uax(o):
        u = jnp.clip((o * scale * 0.5 + 0.5) * (KERNEL - 1), 0.0,
                     float(KERNEL - 1))
        u0 = jnp.clip(jnp.floor(u), 0.0, float(KERNEL - 2))
        return u0.astype(jnp.int32), u - u0

    u0x, fx = uax(ox)
    u0y, fy = uax(oy)
    u0z, fz = uax(oz)
    base = (u0x * KERNEL + u0y) * KERNEL + u0z
    cell_iota = lax.broadcasted_iota(jnp.int32, (1, 1, NCELL), 2)
    e = jnp.zeros((bq, K_NBRS, NCELL), jnp.float32)
    for dxi in (0, 1):
        wx = fx if dxi else (1.0 - fx)
        for dyi in (0, 1):
            wy = fy if dyi else (1.0 - fy)
            for dzi in (0, 1):
                wz = fz if dzi else (1.0 - fz)
                c = base + jnp.int32(dxi * KERNEL * KERNEL + dyi * KERNEL + dzi)
                e = e + jnp.where(cell_iota == c[:, :, None],
                                  (wx * wy * wz * win)[:, :, None], 0.0)
    acc_ref[...] = lax.dot_general(
        e, fg_ref[...].astype(jnp.float32),
        dimension_numbers=(((1,), (1,)), ((0,), (0,))),
        preferred_element_type=jnp.float32)


def _accbuild(qpos, nbrx, nbry, nbrz, maskf, fg):
    n = qpos.shape[0]
    cinp = fg.shape[2]
    bq = 128
    return pl.pallas_call(
        _accbuild_body,
        grid=(n // bq,),
        in_specs=[
            pl.BlockSpec((bq, 3), lambda i: (i, 0)),
            pl.BlockSpec((bq, K_NBRS), lambda i: (i, 0)),
            pl.BlockSpec((bq, K_NBRS), lambda i: (i, 0)),
            pl.BlockSpec((bq, K_NBRS), lambda i: (i, 0)),
            pl.BlockSpec((bq, K_NBRS), lambda i: (i, 0)),
            pl.BlockSpec((bq, K_NBRS, cinp), lambda i: (i, 0, 0)),
        ],
        out_specs=pl.BlockSpec((bq, NCELL, cinp), lambda i: (i, 0, 0)),
        out_shape=jax.ShapeDtypeStruct((n, NCELL, cinp), jnp.float32),
    )(qpos, nbrx, nbry, nbrz, maskf, fg)


def _conv0_body(accO_ref, accF_ref, f0_ref, WcO_ref, WcF_ref, Wd0_ref, b_ref,
                x1_ref, y1_ref):
    oO = jnp.dot(accO_ref[...], WcO_ref[...],
                 preferred_element_type=jnp.float32) + b_ref[0:1, 0:32]
    oF = jnp.dot(accF_ref[...], WcF_ref[...],
                 preferred_element_type=jnp.float32) + b_ref[0:1, 32:64]
    oD = jnp.dot(f0_ref[...], Wd0_ref[...],
                 preferred_element_type=jnp.float32) + b_ref[0:1, 64:96]
    x1 = jnp.concatenate([oO, oF, oD], axis=1)
    x1_ref[...] = x1
    y1_ref[...] = jnp.maximum(x1, 0.0)


def _conv0(accO2, accF2, f0, WcO, WcF, Wd0, b96):
    n = f0.shape[0]
    bq = 256
    return pl.pallas_call(
        _conv0_body,
        grid=(n // bq,),
        in_specs=[
            pl.BlockSpec((bq, accO2.shape[1]), lambda i: (i, 0)),
            pl.BlockSpec((bq, accF2.shape[1]), lambda i: (i, 0)),
            pl.BlockSpec((bq, 4), lambda i: (i, 0)),
            pl.BlockSpec(WcO.shape, lambda i: (0, 0)),
            pl.BlockSpec(WcF.shape, lambda i: (0, 0)),
            pl.BlockSpec(Wd0.shape, lambda i: (0, 0)),
            pl.BlockSpec((1, 96), lambda i: (0, 0)),
        ],
        out_specs=[
            pl.BlockSpec((bq, 96), lambda i: (i, 0)),
            pl.BlockSpec((bq, 96), lambda i: (i, 0)),
        ],
        out_shape=[
            jax.ShapeDtypeStruct((n, 96), jnp.float32),
            jax.ShapeDtypeStruct((n, 96), jnp.float32),
        ],
    )(accO2, accF2, f0, WcO, WcF, Wd0, b96)


def _mid_body(residual, acc_ref, y_ref, x_ref, Wc_ref, Wd_ref, b_ref,
              xn_ref, yn_ref):
    o = jnp.dot(acc_ref[...], Wc_ref[...], preferred_element_type=jnp.float32)
    o = o + jnp.dot(y_ref[...], Wd_ref[...],
                    preferred_element_type=jnp.float32) + b_ref[...]
    if residual:
        o = o + x_ref[...]
    xn_ref[...] = o
    yn_ref[...] = jnp.maximum(o, 0.0)


def _mid(acc2, y, x, Wc, Wd, b, residual):
    n = y.shape[0]
    bq = 256
    cout = Wd.shape[1]
    body = functools.partial(_mid_body, residual)
    return pl.pallas_call(
        body,
        grid=(n // bq,),
        in_specs=[
            pl.BlockSpec((bq, acc2.shape[1]), lambda i: (i, 0)),
            pl.BlockSpec((bq, y.shape[1]), lambda i: (i, 0)),
            pl.BlockSpec((bq, x.shape[1]), lambda i: (i, 0)),
            pl.BlockSpec(Wc.shape, lambda i: (0, 0)),
            pl.BlockSpec(Wd.shape, lambda i: (0, 0)),
            pl.BlockSpec((1, cout), lambda i: (0, 0)),
        ],
        out_specs=[
            pl.BlockSpec((bq, cout), lambda i: (i, 0)),
            pl.BlockSpec((bq, cout), lambda i: (i, 0)),
        ],
        out_shape=[
            jax.ShapeDtypeStruct((n, cout), jnp.float32),
            jax.ShapeDtypeStruct((n, cout), jnp.float32),
        ],
    )(acc2, y, x, Wc, Wd, b)


def _final_body(acc_ref, y_ref, pos2_ref, pos_ref, Wc_ref, Wd_ref, b_ref,
                pn_ref, vn_ref):
    o = jnp.dot(acc_ref[...], Wc_ref[...], preferred_element_type=jnp.float32)
    o = o + jnp.dot(y_ref[...], Wd_ref[...],
                    preferred_element_type=jnp.float32) + b_ref[...]
    pn = pos2_ref[...] + o * (1.0 / 128.0)
    pn_ref[...] = pn
    vn_ref[...] = (pn - pos_ref[...]) * (1.0 / DT)


def _final(acc2, y, pos2, pos, Wc, Wd, b):
    n = y.shape[0]
    bq = 256
    return pl.pallas_call(
        _final_body,
        grid=(n // bq,),
        in_specs=[
            pl.BlockSpec((bq, acc2.shape[1]), lambda i: (i, 0)),
            pl.BlockSpec((bq, y.shape[1]), lambda i: (i, 0)),
            pl.BlockSpec((bq, 3), lambda i: (i, 0)),
            pl.BlockSpec((bq, 3), lambda i: (i, 0)),
            pl.BlockSpec(Wc.shape, lambda i: (0, 0)),
            pl.BlockSpec(Wd.shape, lambda i: (0, 0)),
            pl.BlockSpec((1, 3), lambda i: (0, 0)),
        ],
        out_specs=[
            pl.BlockSpec((bq, 3), lambda i: (i, 0)),
            pl.BlockSpec((bq, 3), lambda i: (i, 0)),
        ],
        out_shape=[
            jax.ShapeDtypeStruct((n, 3), jnp.float32),
            jax.ShapeDtypeStruct((n, 3), jnp.float32),
        ],
    )(acc2, y, pos2, pos, Wc, Wd, b)


def kernel(pos, vel, box, box_feats,
           Wc0f, bc0f, Wc0o, bc0o, Wd0, bd0,
           Wd1, bd1, Wc1, bc1, Wd2, bd2, Wc2, bc2, Wd3, bd3, Wc3, bc3):
    n = pos.shape[0]
    m = box.shape[0]
    gravity = jnp.array([0.0, -9.81, 0.0], dtype=jnp.float32)
    vel2 = vel + DT * gravity
    pos2 = pos + DT * (vel2 + vel) / 2.0
    feats0 = jnp.concatenate([jnp.ones((n, 1), jnp.float32), vel2], axis=1)

    fluid_t = jnp.pad(jnp.concatenate([pos2, feats0], axis=1),
                      ((0, 0), (0, 9)))
    box_t = jnp.pad(jnp.concatenate([box, box_feats], axis=1),
                    ((0, 0), (0, 10)))
    pos2T = jnp.pad(pos2.T, ((0, 5), (0, 0)))
    boxT = jnp.pad(box.T, ((0, 5), (0, 0)))

    idxF, maskF = _knn(pos2, pos2T, n, True)
    idxO, maskO = _knn(pos2, boxT, m, False)
    idxF_flat = idxF.reshape(-1)
    idxO_flat = idxO.reshape(-1)
    gF = _sc_gather(fluid_t, idxF_flat)
    gO = _sc_gather(box_t, idxO_flat)
    nbrxF = gF[:, 0].reshape(n, K_NBRS)
    nbryF = gF[:, 1].reshape(n, K_NBRS)
    nbrzF = gF[:, 2].reshape(n, K_NBRS)
    nbrxO = gO[:, 0].reshape(n, K_NBRS)
    nbryO = gO[:, 1].reshape(n, K_NBRS)
    nbrzO = gO[:, 2].reshape(n, K_NBRS)

    fgF = gF[:, 3:7].reshape(n, K_NBRS, 4)
    fgO = gO[:, 3:6].reshape(n, K_NBRS, 3)
    accF = _accbuild(pos2, nbrxF, nbryF, nbrzF, maskF, fgF).reshape(n, NCELL * 4)
    accO = _accbuild(pos2, nbrxO, nbryO, nbrzO, maskO, fgO).reshape(n, NCELL * 3)
    WcF_flat = Wc0f.reshape(NCELL * 4, 32)
    WcO_flat = Wc0o.reshape(NCELL * 3, 32)
    b96 = jnp.concatenate([bc0o, bc0f, bd0]).reshape(1, 96)
    x1, y1 = _conv0(accO, accF, feats0, WcO_flat, WcF_flat, Wd0, b96)

    fg1 = _sc_gather(y1, idxF_flat).reshape(n, K_NBRS, 96)
    acc1 = _accbuild(pos2, nbrxF, nbryF, nbrzF, maskF, fg1).reshape(n, NCELL * 96)
    x2, y2 = _mid(acc1, y1, y1, Wc1.reshape(NCELL * 96, 64), Wd1,
                  (bc1 + bd1).reshape(1, 64), residual=False)

    fg2 = _sc_gather(y2, idxF_flat).reshape(n, K_NBRS, 64)
    acc2 = _accbuild(pos2, nbrxF, nbryF, nbrzF, maskF, fg2).reshape(n, NCELL * 64)
    x3, y3 = _mid(acc2, y2, x2, Wc2.reshape(NCELL * 64, 64), Wd2,
                  (bc2 + bd2).reshape(1, 64), residual=True)

    fg3 = _sc_gather(y3, idxF_flat).reshape(n, K_NBRS, 64)
    acc3 = _accbuild(pos2, nbrxF, nbryF, nbrzF, maskF, fg3).reshape(n, NCELL * 64)
    pos_new, vel_new = _final(acc3, y3, pos2, pos, Wc3.reshape(NCELL * 64, 3),
                              Wd3, (bc3 + bd3).reshape(1, 3))
    return pos_new, vel_new
